# Initial kernel scaffold; baseline (speedup 1.0000x reference)
import jax
import jax.numpy as jnp
from jax.experimental import pallas as pl


def kernel(x, edge_index, W1l, b1l, W1r, W2l, b2l, W2r, W3l, b3l, W3r, W4l, b4l, W4r, W5l, b5l, W5r, W6l, b6l, W6r, Wn, bn):
    raise NotImplementedError("write your pallas kernel here")


# speedup vs baseline: 4.5413x; 4.5413x over previous
import functools

import jax
import jax.numpy as jnp
from jax import lax
from jax.experimental import pallas as pl
from jax.experimental.pallas import tpu as pltpu
from jax.experimental.pallas import tpu_sc as plsc

N = 10000
E = 320000
NWORK = 32
EPW = E // NWORK
EB = 128
NB = EPW // EB
TAIL = EPW - NB * EB
DC = 128
RQ = 624
EX = N - 16 * RQ
ZR = 208

_f32 = jnp.float32


def _mesh():
    return plsc.VectorSubcoreMesh(core_axis_name="c", subcore_axis_name="s")


def _make_agg(nc):
    @functools.partial(
        pl.kernel,
        out_type=jax.ShapeDtypeStruct((2, nc, N, DC), _f32),
        mesh=_mesh(),
        scratch_types=[
            pltpu.VMEM((EB,), jnp.int32),
            pltpu.VMEM((EB,), jnp.int32),
            pltpu.VMEM((EB, DC), _f32),
            pltpu.VMEM((TAIL,), jnp.int32),
            pltpu.VMEM((TAIL,), jnp.int32),
            pltpu.VMEM((TAIL, DC), _f32),
            pltpu.VMEM((ZR, DC), _f32),
            pltpu.VMEM_SHARED((N, DC), _f32),
            pltpu.SemaphoreType.DMA,
        ],
    )
    def agg(*refs):
        tables = refs[:nc]
        src_hbm, dst_hbm, out_hbm = refs[nc], refs[nc + 1], refs[nc + 2]
        (src_v, dst_v, rows_v, src_t, dst_t, rows_t, zb, acc, sem) = refs[nc + 3:]
        cid = lax.axis_index("c")
        sid = lax.axis_index("s")
        wid = cid * 16 + sid
        ebase = wid * EPW
        zeros = jnp.zeros((16,), _f32)

        def zrow(i, _):
            def zcol(j, __):
                zb[i, pl.ds(j * 16, 16)] = zeros
                return 0

            return lax.fori_loop(0, DC // 16, zcol, 0)

        lax.fori_loop(0, ZR, zrow, 0)

        for c in range(nc):
            for r in range(RQ // ZR):
                pltpu.sync_copy(zb, acc.at[pl.ds(sid * RQ + r * ZR, ZR), :])

            @pl.when(sid == 15)
            def _():
                pltpu.sync_copy(zb.at[pl.ds(0, EX), :],
                                acc.at[pl.ds(16 * RQ, EX), :])

            plsc.subcore_barrier()

            def blk(b, _):
                e0 = ebase + b * EB
                pltpu.sync_copy(src_hbm.at[pl.ds(e0, EB)], src_v)
                pltpu.sync_copy(dst_hbm.at[pl.ds(e0, EB)], dst_v)
                pltpu.async_copy(tables[c].at[src_v], rows_v, sem).wait()
                pltpu.sync_copy(rows_v, acc.at[dst_v], add=True)
                return 0

            lax.fori_loop(0, NB, blk, 0)
            e0 = ebase + NB * EB
            pltpu.sync_copy(src_hbm.at[pl.ds(e0, TAIL)], src_t)
            pltpu.sync_copy(dst_hbm.at[pl.ds(e0, TAIL)], dst_t)
            pltpu.async_copy(tables[c].at[src_t], rows_t, sem).wait()
            pltpu.sync_copy(rows_t, acc.at[dst_t], add=True)
            plsc.subcore_barrier()
            pltpu.sync_copy(
                acc.at[pl.ds(sid * RQ, RQ), :],
                out_hbm.at[cid, c, pl.ds(sid * RQ, RQ), :],
            )

            @pl.when(sid == 15)
            def _():
                pltpu.sync_copy(acc.at[pl.ds(16 * RQ, EX), :],
                                out_hbm.at[cid, c, pl.ds(16 * RQ, EX), :])

    return agg


BN = 1000


def _relu(x):
    return jnp.maximum(x, 0.0)


def _leaky(x):
    return jnp.where(x >= 0.0, x, 0.1 * x)


def _mm(h, w):
    return lax.dot_general(h, w, (((1,), (1,)), ((), ())),
                           preferred_element_type=_f32)


def _pad_tables(z, bn):
    d = z.shape[1]
    out = []
    for c0 in range(0, d, DC):
        w = min(DC, d - c0)
        t = z[:, c0:c0 + w]
        if w < DC:
            t = jnp.concatenate([t, jnp.zeros((bn, DC - w), _f32)], axis=1)
        out.append(t)
    return out


def _premm_ones_op(h, w):
    din = h.shape[1]
    dout = w.shape[0]

    def body(h_ref, w_ref, o_ref):
        z = _mm(h_ref[...], w_ref[...])
        o_ref[...] = jnp.concatenate(
            [z, jnp.ones((BN, DC - dout), _f32)], axis=1)

    return pl.pallas_call(
        body,
        grid=(N // BN,),
        in_specs=[
            pl.BlockSpec((BN, din), lambda i: (i, 0)),
            pl.BlockSpec((dout, din), lambda i: (0, 0)),
        ],
        out_specs=pl.BlockSpec((BN, DC), lambda i: (i, 0)),
        out_shape=jax.ShapeDtypeStruct((N, DC), _f32),
    )(h, w)


def _invdeg_op(P1):
    def body(p_ref, o_ref):
        s = p_ref[0, 0, :, 64] + p_ref[1, 0, :, 64]
        o_ref[...] = (1.0 / jnp.maximum(s, 1.0))[:, None]

    return pl.pallas_call(
        body,
        grid=(N // BN,),
        in_specs=[pl.BlockSpec((2, 1, BN, DC), lambda i: (0, 0, i, 0))],
        out_specs=pl.BlockSpec((BN, 1), lambda i: (i, 0)),
        out_shape=jax.ShapeDtypeStruct((N, 1), _f32),
    )(P1)


def _combine_op(P, invdeg, h, act, *, d_agg, pre, bl, Wr, Wl=None, emits=()):
    _, nc_in, _, _ = P.shape
    din = h.shape[1]
    dout = Wr.shape[0]
    widths = [min(DC, d_agg - c * DC) for c in range(nc_in)]

    def body(p_ref, iv_ref, h_ref, bl_ref, wr_ref, *rest):
        refs = list(rest)
        wl_ref = None if pre else refs.pop(0)
        emit_w = [refs.pop(0) for e in emits if e[0] == "mm"]
        o_refs = refs
        iv = iv_ref[...]
        hv = h_ref[...]
        acc = _mm(hv, wr_ref[...]) + bl_ref[...]
        if pre:
            cols = []
            for c in range(nc_in):
                mc = (p_ref[0, c, :, :widths[c]]
                      + p_ref[1, c, :, :widths[c]]) * iv
                cols.append(acc[:, c * DC:c * DC + widths[c]] + mc)
            acc = jnp.concatenate(cols, axis=1) if nc_in > 1 else cols[0]
        else:
            wl = wl_ref[...]
            for c in range(nc_in):
                mc = (p_ref[0, c, :, :widths[c]]
                      + p_ref[1, c, :, :widths[c]]) * iv
                acc = acc + _mm(mc, wl[:, c * DC:c * DC + widths[c]])
        out = act(acc)
        o_refs[0][...] = out
        oi = 1
        wi = 0
        for e in emits:
            if e[0] == "mm":
                z = _mm(out, emit_w[wi][...])
                wi += 1
            else:
                z = out
            for t in _pad_tables(z, BN):
                o_refs[oi][...] = t
                oi += 1

    in_specs = [
        pl.BlockSpec((2, nc_in, BN, DC), lambda i: (0, 0, i, 0)),
        pl.BlockSpec((BN, 1), lambda i: (i, 0)),
        pl.BlockSpec((BN, din), lambda i: (i, 0)),
        pl.BlockSpec((1, dout), lambda i: (0, 0)),
        pl.BlockSpec((dout, din), lambda i: (0, 0)),
    ]
    args = [P, invdeg, h, bl.reshape(1, dout), Wr]
    if not pre:
        in_specs.append(pl.BlockSpec(Wl.shape, lambda i: (0, 0)))
        args.append(Wl)
    out_specs = [pl.BlockSpec((BN, dout), lambda i: (i, 0))]
    out_shape = [jax.ShapeDtypeStruct((N, dout), _f32)]
    for e in emits:
        if e[0] == "mm":
            W = e[1]
            in_specs.append(pl.BlockSpec(W.shape, lambda i: (0, 0)))
            args.append(W)
            ncols = W.shape[0]
        else:
            ncols = dout
        ntab = -(-ncols // DC)
        out_specs += [pl.BlockSpec((BN, DC), lambda i: (i, 0))] * ntab
        out_shape += [jax.ShapeDtypeStruct((N, DC), _f32)] * ntab

    return pl.pallas_call(
        body,
        grid=(N // BN,),
        in_specs=in_specs,
        out_specs=out_specs,
        out_shape=out_shape,
    )(*args)


def _final_op(hf, wn, bn_):
    K = hf.shape[1]
    KB = K // 8

    def body(h_ref, w_ref, b_ref, o_ref):
        k = pl.program_id(0)

        @pl.when(k == 0)
        def _():
            o_ref[...] = jnp.broadcast_to(b_ref[...], o_ref.shape)

        o_ref[...] += _mm(h_ref[...], w_ref[...])

        @pl.when(k == 7)
        def _():
            o_ref[...] = _relu(o_ref[...])

    return pl.pallas_call(
        body,
        grid=(8,),
        in_specs=[
            pl.BlockSpec((100, KB), lambda k: (0, k)),
            pl.BlockSpec((100, KB), lambda k: (0, k)),
            pl.BlockSpec((1, 100), lambda k: (0, 0)),
        ],
        out_specs=pl.BlockSpec((100, 100), lambda k: (0, 0)),
        out_shape=jax.ShapeDtypeStruct((100, 100), _f32),
    )(hf, wn, bn_.reshape(1, 100))


_agg1 = _make_agg(1)
_agg2 = _make_agg(2)
_agg4 = _make_agg(4)


@jax.jit
def kernel(x, edge_index, W1l, b1l, W1r, W2l, b2l, W2r, W3l, b3l, W3r,
           W4l, b4l, W4r, W5l, b5l, W5r, W6l, b6l, W6r, Wn, bn):
    src = edge_index[0]
    dst = edge_index[1]

    z1 = _premm_ones_op(x, W1l)
    P1 = _agg1(z1, src, dst)
    invdeg = _invdeg_op(P1)
    h1, z2 = _combine_op(P1, invdeg, x, _relu, d_agg=64, pre=True, bl=b1l,
                         Wr=W1r, emits=(("mm", W2l),))

    P2 = _agg1(z2, src, dst)
    h2, h2t = _combine_op(P2, invdeg, h1, _leaky, d_agg=64, pre=True,
                          bl=b2l, Wr=W2r, emits=(("copy",),))

    P3 = _agg1(h2t, src, dst)
    h3, z4a, z4b = _combine_op(P3, invdeg, h2, _relu, d_agg=64, pre=False,
                               bl=b3l, Wr=W3r, Wl=W3l, emits=(("mm", W4l),))

    P4 = _agg2(z4a, z4b, src, dst)
    h4, h4a, h4b = _combine_op(P4, invdeg, h3, _leaky, d_agg=256, pre=True,
                               bl=b4l, Wr=W4r, emits=(("copy",),))

    P5 = _agg2(h4a, h4b, src, dst)
    h5, z6a, z6b, z6c, z6d = _combine_op(
        P5, invdeg, h4, _relu, d_agg=256, pre=False, bl=b5l, Wr=W5r, Wl=W5l,
        emits=(("mm", W6l),))

    P6 = _agg4(z6a, z6b, z6c, z6d, src, dst)
    h6 = _combine_op(P6, invdeg, h5, _leaky, d_agg=512, pre=True, bl=b6l,
                     Wr=W6r)[0]

    hf = h6.reshape(100, 512 * 100)
    return _final_op(hf, Wn, bn)
